# baseline (speedup 1.0000x reference)
import functools

import jax
import jax.numpy as jnp
from jax import lax
from jax.experimental import pallas as pl
from jax.experimental.pallas import tpu as pltpu
from jax.experimental.pallas import tpu_sc as plsc

N = 10000
D = 128
E = 320000
C = 96
NC = 2
NS = 16
NBUF = 3
PD = 2
NR = 6
IPD = 3
CH = (66, 150)
TOT_CHUNKS = NS * (CH[0] + CH[1])
E_PAD = TOT_CHUNKS * C
N_PAD = 10112
RPT = N_PAD // NS


def _spmm_sc(h, src2, dst2, w2):
    mesh = plsc.VectorSubcoreMesh(core_axis_name="c", subcore_axis_name="s")

    @functools.partial(
        pl.kernel,
        out_type=jax.ShapeDtypeStruct((NC, N_PAD, D), jnp.float32),
        mesh=mesh,
        scratch_types=[
            pltpu.VMEM((NR, C), jnp.int32),
            pltpu.VMEM((NR, C), jnp.int32),
            pltpu.VMEM((NR, C), jnp.float32),
            pltpu.VMEM((NBUF, C, D), jnp.float32),
            pltpu.VMEM_SHARED((N_PAD, D), jnp.float32),
            pltpu.SemaphoreType.DMA,
            pltpu.SemaphoreType.DMA,
            pltpu.SemaphoreType.DMA,
            pltpu.SemaphoreType.DMA,
            pltpu.SemaphoreType.DMA,
            pltpu.SemaphoreType.DMA,
            pltpu.SemaphoreType.DMA,
            pltpu.SemaphoreType.DMA,
            pltpu.SemaphoreType.DMA,
            pltpu.SemaphoreType.DMA,
            pltpu.SemaphoreType.DMA,
            pltpu.SemaphoreType.DMA,
        ],
    )
    def k(h_hbm, src_hbm, dst_hbm, w_hbm, out_hbm,
          src_v, dst_v, w_v, rows_v, accum,
          si0, si1, si2, si3, si4, si5, sg0, sg1, sg2, ss0, ss1, ss2):
        sem_i = (si0, si1, si2, si3, si4, si5)
        sem_g = (sg0, sg1, sg2)
        sem_s = (ss0, ss1, ss2)
        cid = lax.axis_index("c")
        sid = lax.axis_index("s")
        ch = jnp.where(cid == 0, CH[0], CH[1])
        cbase = jnp.where(cid == 0, sid * CH[0], NS * CH[0] + sid * CH[1])

        def issue_idx(ci, s):
            pltpu.async_copy(src_hbm.at[cbase + ci], src_v.at[s], sem_i[s])
            pltpu.async_copy(dst_hbm.at[cbase + ci], dst_v.at[s], sem_i[s])
            pltpu.async_copy(w_hbm.at[cbase + ci], w_v.at[s], sem_i[s])

        def wait_idx(ci, s):
            pltpu.make_async_copy(src_hbm.at[cbase + ci], src_v.at[s],
                                  sem_i[s]).wait()
            pltpu.make_async_copy(dst_hbm.at[cbase + ci], dst_v.at[s],
                                  sem_i[s]).wait()
            pltpu.make_async_copy(w_hbm.at[cbase + ci], w_v.at[s],
                                  sem_i[s]).wait()

        def issue_gather(s, b):
            pltpu.async_copy(h_hbm.at[src_v.at[s]], rows_v.at[b], sem_g[b])

        def wait_gather(s, b):
            pltpu.make_async_copy(h_hbm.at[src_v.at[s]], rows_v.at[b],
                                  sem_g[b]).wait()

        def issue_scatter(s, b):
            pltpu.async_copy(rows_v.at[b], accum.at[dst_v.at[s]], sem_s[b],
                             add=True)

        def wait_scatter(s, b):
            pltpu.make_async_copy(rows_v.at[b], accum.at[dst_v.at[s]],
                                  sem_s[b]).wait()

        for k0 in range(IPD):
            issue_idx(k0, k0)

        def zrow(r, _):
            for j in range(D // 16):
                rows_v[0, r, pl.ds(j * 16, 16)] = jnp.zeros((16,), jnp.float32)
            return 0
        lax.fori_loop(0, C, zrow, 0)
        base = sid * RPT
        for kblk in range(RPT // C):
            pltpu.sync_copy(rows_v.at[0], accum.at[pl.ds(base + kblk * C, C)])
        rem = RPT - (RPT // C) * C
        if rem:
            pltpu.sync_copy(rows_v.at[0].at[pl.ds(0, rem)],
                            accum.at[pl.ds(base + (RPT // C) * C, rem)])

        for k0 in range(PD):
            wait_idx(k0, k0)
            issue_gather(k0, k0)

        plsc.subcore_barrier()

        def super_body(cs, _):
            for u in range(NR):
                b = u % NBUF
                ci = cs * NR + u
                wait_gather(u, b)

                def scale_group(g, _):
                    wg = w_v[u, pl.ds(g * 16, 16)]
                    for l in range(16):
                        ws = wg[l]
                        r = g * 16 + l
                        for j in range(D // 16):
                            sl = pl.ds(j * 16, 16)
                            rows_v[b, r, sl] = rows_v[b, r, sl] * ws
                    return 0
                lax.fori_loop(0, C // 16, scale_group, 0)

                issue_scatter(u, b)

                ci2 = ci + PD
                b2 = (u + PD) % NBUF
                s2 = (u + PD) % NR

                @pl.when(ci2 < ch)
                def _():
                    @pl.when(ci2 >= NBUF)
                    def _():
                        wait_scatter((u + PD - NBUF) % NR, b2)
                    wait_idx(ci2, s2)
                    issue_gather(s2, b2)

                ci3 = ci + IPD
                s3 = (u + IPD) % NR

                @pl.when(ci3 < ch)
                def _():
                    issue_idx(ci3, s3)
            return 0

        lax.fori_loop(0, ch // NR, super_body, 0)

        for k0 in range(NBUF):
            s = NR - NBUF + k0
            wait_scatter(s, s % NBUF)
        plsc.subcore_barrier()

        pltpu.sync_copy(accum.at[pl.ds(base, RPT)],
                        out_hbm.at[cid, pl.ds(base, RPT)])

    return k(h, src2, dst2, w2)


R_BLK = 1000


def _fuse1(x, p0, p1, w0, w1):
    def body(x_b, p0_b, p1_b, w0_b, w1_b, h1_b, acc_b):
        h1 = p0_b[...] + p1_b[...]
        h1_b[...] = h1
        acc_b[...] = (
            jnp.dot(x_b[...], w0_b[...], preferred_element_type=jnp.float32)
            + jnp.dot(h1, w1_b[...], preferred_element_type=jnp.float32)
        )

    row_spec = pl.BlockSpec((R_BLK, D), lambda i: (i, 0))
    w_spec = pl.BlockSpec((D, D), lambda i: (0, 0))
    return pl.pallas_call(
        body,
        grid=(N // R_BLK,),
        in_specs=[row_spec, row_spec, row_spec, w_spec, w_spec],
        out_specs=[row_spec, row_spec],
        out_shape=[
            jax.ShapeDtypeStruct((N, D), jnp.float32),
            jax.ShapeDtypeStruct((N, D), jnp.float32),
        ],
    )(x, p0, p1, w0, w1)


def _fuse2(acc, q0, q1, w2, bias):
    def body(acc_b, q0_b, q1_b, w2_b, b_b, out_b):
        h2 = q0_b[...] + q1_b[...]
        out_b[...] = (
            acc_b[...]
            + jnp.dot(h2, w2_b[...], preferred_element_type=jnp.float32)
            + b_b[...]
        )

    row_spec = pl.BlockSpec((R_BLK, D), lambda i: (i, 0))
    w_spec = pl.BlockSpec((D, D), lambda i: (0, 0))
    b_spec = pl.BlockSpec((1, D), lambda i: (0, 0))
    return pl.pallas_call(
        body,
        grid=(N // R_BLK,),
        in_specs=[row_spec, row_spec, row_spec, w_spec, b_spec],
        out_specs=row_spec,
        out_shape=jax.ShapeDtypeStruct((N, D), jnp.float32),
    )(acc, q0, q1, w2, bias)


def kernel(x, edge_index, edge_weight, shared_weight, output_bias):
    src = edge_index[1].astype(jnp.int32)
    dst = edge_index[0].astype(jnp.int32)
    w = edge_weight.astype(jnp.float32)
    pad = E_PAD - E
    src = jnp.concatenate([src, jnp.zeros((pad,), jnp.int32)])
    dst = jnp.concatenate([dst, jnp.zeros((pad,), jnp.int32)])
    w = jnp.concatenate([w, jnp.zeros((pad,), jnp.float32)])
    src2 = src.reshape(TOT_CHUNKS, C)
    dst2 = dst.reshape(TOT_CHUNKS, C)
    w2 = w.reshape(TOT_CHUNKS, C)

    w0 = shared_weight[:D]
    w1 = shared_weight[D:2 * D]
    w2b = shared_weight[2 * D:]
    bias = output_bias.reshape(1, D)

    p = _spmm_sc(x, src2, dst2, w2)
    h1, acc = _fuse1(x, p[0], p[1], w0, w1)
    q = _spmm_sc(h1, src2, dst2, w2)
    return _fuse2(acc, q[0], q[1], w2b, bias)


# speedup vs baseline: 1.0317x; 1.0317x over previous
import functools

import jax
import jax.numpy as jnp
from jax import lax
from jax.experimental import pallas as pl
from jax.experimental.pallas import tpu as pltpu
from jax.experimental.pallas import tpu_sc as plsc

N = 10000
D = 128
E = 320000
C = 96
NC = 2
NS = 16
NBUF = 3
PD = 2
NR = 6
IPD = 3
CH = (150, 66)
TOT_CHUNKS = NS * (CH[0] + CH[1])
E_PAD = TOT_CHUNKS * C
N_PAD = 10112
RPT = N_PAD // NS


def _spmm_sc(h, src2, dst2, w2):
    mesh = plsc.VectorSubcoreMesh(core_axis_name="c", subcore_axis_name="s")

    @functools.partial(
        pl.kernel,
        out_type=jax.ShapeDtypeStruct((NC, N_PAD, D), jnp.float32),
        mesh=mesh,
        scratch_types=[
            pltpu.VMEM((NR, C), jnp.int32),
            pltpu.VMEM((NR, C), jnp.int32),
            pltpu.VMEM((NR, C), jnp.float32),
            pltpu.VMEM((NBUF, C, D), jnp.float32),
            pltpu.VMEM_SHARED((N_PAD, D), jnp.float32),
            pltpu.SemaphoreType.DMA,
            pltpu.SemaphoreType.DMA,
            pltpu.SemaphoreType.DMA,
            pltpu.SemaphoreType.DMA,
            pltpu.SemaphoreType.DMA,
            pltpu.SemaphoreType.DMA,
            pltpu.SemaphoreType.DMA,
            pltpu.SemaphoreType.DMA,
            pltpu.SemaphoreType.DMA,
            pltpu.SemaphoreType.DMA,
            pltpu.SemaphoreType.DMA,
            pltpu.SemaphoreType.DMA,
        ],
    )
    def k(h_hbm, src_hbm, dst_hbm, w_hbm, out_hbm,
          src_v, dst_v, w_v, rows_v, accum,
          si0, si1, si2, si3, si4, si5, sg0, sg1, sg2, ss0, ss1, ss2):
        sem_i = (si0, si1, si2, si3, si4, si5)
        sem_g = (sg0, sg1, sg2)
        sem_s = (ss0, ss1, ss2)
        cid = lax.axis_index("c")
        sid = lax.axis_index("s")
        ch = jnp.where(cid == 0, CH[0], CH[1])
        cbase = jnp.where(cid == 0, sid * CH[0], NS * CH[0] + sid * CH[1])

        def issue_idx(ci, s):
            pltpu.async_copy(src_hbm.at[cbase + ci], src_v.at[s], sem_i[s])
            pltpu.async_copy(dst_hbm.at[cbase + ci], dst_v.at[s], sem_i[s])
            pltpu.async_copy(w_hbm.at[cbase + ci], w_v.at[s], sem_i[s])

        def wait_idx(ci, s):
            pltpu.make_async_copy(src_hbm.at[cbase + ci], src_v.at[s],
                                  sem_i[s]).wait()
            pltpu.make_async_copy(dst_hbm.at[cbase + ci], dst_v.at[s],
                                  sem_i[s]).wait()
            pltpu.make_async_copy(w_hbm.at[cbase + ci], w_v.at[s],
                                  sem_i[s]).wait()

        def issue_gather(s, b):
            pltpu.async_copy(h_hbm.at[src_v.at[s]], rows_v.at[b], sem_g[b])

        def wait_gather(s, b):
            pltpu.make_async_copy(h_hbm.at[src_v.at[s]], rows_v.at[b],
                                  sem_g[b]).wait()

        def issue_scatter(s, b):
            pltpu.async_copy(rows_v.at[b], accum.at[dst_v.at[s]], sem_s[b],
                             add=True)

        def wait_scatter(s, b):
            pltpu.make_async_copy(rows_v.at[b], accum.at[dst_v.at[s]],
                                  sem_s[b]).wait()

        for k0 in range(IPD):
            issue_idx(k0, k0)

        def zrow(r, _):
            for j in range(D // 16):
                rows_v[0, r, pl.ds(j * 16, 16)] = jnp.zeros((16,), jnp.float32)
            return 0
        lax.fori_loop(0, C, zrow, 0)
        base = sid * RPT
        for kblk in range(RPT // C):
            pltpu.sync_copy(rows_v.at[0], accum.at[pl.ds(base + kblk * C, C)])
        rem = RPT - (RPT // C) * C
        if rem:
            pltpu.sync_copy(rows_v.at[0].at[pl.ds(0, rem)],
                            accum.at[pl.ds(base + (RPT // C) * C, rem)])

        for k0 in range(PD):
            wait_idx(k0, k0)
            issue_gather(k0, k0)

        plsc.subcore_barrier()

        def super_body(cs, _):
            for u in range(NR):
                b = u % NBUF
                ci = cs * NR + u
                wait_gather(u, b)

                def scale_group(g, _):
                    wg = w_v[u, pl.ds(g * 16, 16)]
                    for l in range(16):
                        ws = wg[l]
                        r = g * 16 + l
                        for j in range(D // 16):
                            sl = pl.ds(j * 16, 16)
                            rows_v[b, r, sl] = rows_v[b, r, sl] * ws
                    return 0
                lax.fori_loop(0, C // 16, scale_group, 0)

                issue_scatter(u, b)

                ci2 = ci + PD
                b2 = (u + PD) % NBUF
                s2 = (u + PD) % NR

                @pl.when(ci2 < ch)
                def _():
                    @pl.when(ci2 >= NBUF)
                    def _():
                        wait_scatter((u + PD - NBUF) % NR, b2)
                    wait_idx(ci2, s2)
                    issue_gather(s2, b2)

                ci3 = ci + IPD
                s3 = (u + IPD) % NR

                @pl.when(ci3 < ch)
                def _():
                    issue_idx(ci3, s3)
            return 0

        lax.fori_loop(0, ch // NR, super_body, 0)

        for k0 in range(NBUF):
            s = NR - NBUF + k0
            wait_scatter(s, s % NBUF)
        plsc.subcore_barrier()

        pltpu.sync_copy(accum.at[pl.ds(base, RPT)],
                        out_hbm.at[cid, pl.ds(base, RPT)])

    return k(h, src2, dst2, w2)


R_BLK = 1000


def _fuse1(x, p0, p1, w0, w1):
    def body(x_b, p0_b, p1_b, w0_b, w1_b, h1_b, acc_b):
        h1 = p0_b[...] + p1_b[...]
        h1_b[...] = h1
        acc_b[...] = (
            jnp.dot(x_b[...], w0_b[...], preferred_element_type=jnp.float32)
            + jnp.dot(h1, w1_b[...], preferred_element_type=jnp.float32)
        )

    row_spec = pl.BlockSpec((R_BLK, D), lambda i: (i, 0))
    w_spec = pl.BlockSpec((D, D), lambda i: (0, 0))
    return pl.pallas_call(
        body,
        grid=(N // R_BLK,),
        in_specs=[row_spec, row_spec, row_spec, w_spec, w_spec],
        out_specs=[row_spec, row_spec],
        out_shape=[
            jax.ShapeDtypeStruct((N, D), jnp.float32),
            jax.ShapeDtypeStruct((N, D), jnp.float32),
        ],
    )(x, p0, p1, w0, w1)


def _fuse2(acc, q0, q1, w2, bias):
    def body(acc_b, q0_b, q1_b, w2_b, b_b, out_b):
        h2 = q0_b[...] + q1_b[...]
        out_b[...] = (
            acc_b[...]
            + jnp.dot(h2, w2_b[...], preferred_element_type=jnp.float32)
            + b_b[...]
        )

    row_spec = pl.BlockSpec((R_BLK, D), lambda i: (i, 0))
    w_spec = pl.BlockSpec((D, D), lambda i: (0, 0))
    b_spec = pl.BlockSpec((1, D), lambda i: (0, 0))
    return pl.pallas_call(
        body,
        grid=(N // R_BLK,),
        in_specs=[row_spec, row_spec, row_spec, w_spec, b_spec],
        out_specs=row_spec,
        out_shape=jax.ShapeDtypeStruct((N, D), jnp.float32),
    )(acc, q0, q1, w2, bias)


def kernel(x, edge_index, edge_weight, shared_weight, output_bias):
    src = edge_index[1].astype(jnp.int32)
    dst = edge_index[0].astype(jnp.int32)
    w = edge_weight.astype(jnp.float32)
    pad = E_PAD - E
    src = jnp.concatenate([src, jnp.zeros((pad,), jnp.int32)])
    dst = jnp.concatenate([dst, jnp.zeros((pad,), jnp.int32)])
    w = jnp.concatenate([w, jnp.zeros((pad,), jnp.float32)])
    src2 = src.reshape(TOT_CHUNKS, C)
    dst2 = dst.reshape(TOT_CHUNKS, C)
    w2 = w.reshape(TOT_CHUNKS, C)

    w0 = shared_weight[:D]
    w1 = shared_weight[D:2 * D]
    w2b = shared_weight[2 * D:]
    bias = output_bias.reshape(1, D)

    p = _spmm_sc(x, src2, dst2, w2)
    h1, acc = _fuse1(x, p[0], p[1], w0, w1)
    q = _spmm_sc(h1, src2, dst2, w2)
    return _fuse2(acc, q[0], q[1], w2b, bias)


# speedup vs baseline: 1.1760x; 1.1399x over previous
import functools

import jax
import jax.numpy as jnp
from jax import lax
from jax.experimental import pallas as pl
from jax.experimental.pallas import tpu as pltpu
from jax.experimental.pallas import tpu_sc as plsc

N = 10000
D = 128
E = 320000
C = 128
NC = 2
NS = 16
NW = NC * NS
CHUNKS_TOTAL = -(-E // (C * NW))
E_PAD = CHUNKS_TOTAL * C * NW
PER_W = E_PAD // NW
CHUNKS = PER_W // C
N_PAD = 10240
RPT = N_PAD // NS


def _spmm_sc(h, src, dst, w):
    mesh = plsc.VectorSubcoreMesh(core_axis_name="c", subcore_axis_name="s")

    @functools.partial(
        pl.kernel,
        out_type=jax.ShapeDtypeStruct((NC, N_PAD, D), jnp.float32),
        mesh=mesh,
        scratch_types=[
            pltpu.VMEM((C,), jnp.int32),
            pltpu.VMEM((C,), jnp.int32),
            pltpu.VMEM((C + 16,), jnp.float32),
            pltpu.VMEM((C, D), jnp.float32),
            pltpu.VMEM_SHARED((N_PAD, D), jnp.float32),
            pltpu.SemaphoreType.DMA,
        ],
    )
    def k(h_hbm, src_hbm, dst_hbm, w_hbm, out_hbm,
          idx_v, dst_v, w_v, rows_v, accum, sem):
        cid = lax.axis_index("c")
        sid = lax.axis_index("s")
        wid = sid * NC + cid

        def zrow(r, _):
            for j in range(D // 16):
                rows_v[r, pl.ds(j * 16, 16)] = jnp.zeros((16,), jnp.float32)
            return 0
        lax.fori_loop(0, C, zrow, 0)
        base = sid * RPT
        for kblk in range(RPT // C):
            pltpu.sync_copy(rows_v, accum.at[pl.ds(base + kblk * C, C)])
        plsc.subcore_barrier()

        ebase = wid * PER_W

        def chunk_body(ci, _):
            off = ebase + ci * C
            pltpu.sync_copy(src_hbm.at[pl.ds(off, C)], idx_v)
            pltpu.sync_copy(dst_hbm.at[pl.ds(off, C)], dst_v)
            pltpu.sync_copy(w_hbm.at[pl.ds(off, C)], w_v.at[pl.ds(0, C)])
            pltpu.async_copy(h_hbm.at[idx_v], rows_v, sem).wait()

            def row_body(r, _):
                ws = w_v[pl.ds(r, 16)][0]
                for j in range(D // 16):
                    sl = pl.ds(j * 16, 16)
                    rows_v[r, sl] = rows_v[r, sl] * ws
                return 0
            lax.fori_loop(0, C, row_body, 0)

            pltpu.sync_copy(rows_v, accum.at[dst_v], add=True)
            return 0

        lax.fori_loop(0, CHUNKS, chunk_body, 0)
        plsc.subcore_barrier()

        pltpu.sync_copy(accum.at[pl.ds(base, RPT)],
                        out_hbm.at[cid, pl.ds(base, RPT)])

    return k(h, src, dst, w)


R_BLK = 1000


def _fuse1(x, p0, p1, w0, w1):
    def body(x_b, p0_b, p1_b, w0_b, w1_b, h1_b, acc_b):
        h1 = p0_b[...] + p1_b[...]
        h1_b[...] = h1
        acc_b[...] = (
            jnp.dot(x_b[...], w0_b[...], preferred_element_type=jnp.float32)
            + jnp.dot(h1, w1_b[...], preferred_element_type=jnp.float32)
        )

    row_spec = pl.BlockSpec((R_BLK, D), lambda i: (i, 0))
    w_spec = pl.BlockSpec((D, D), lambda i: (0, 0))
    return pl.pallas_call(
        body,
        grid=(N // R_BLK,),
        in_specs=[row_spec, row_spec, row_spec, w_spec, w_spec],
        out_specs=[row_spec, row_spec],
        out_shape=[
            jax.ShapeDtypeStruct((N, D), jnp.float32),
            jax.ShapeDtypeStruct((N, D), jnp.float32),
        ],
    )(x, p0, p1, w0, w1)


def _fuse2(acc, q0, q1, w2, bias):
    def body(acc_b, q0_b, q1_b, w2_b, b_b, out_b):
        h2 = q0_b[...] + q1_b[...]
        out_b[...] = (
            acc_b[...]
            + jnp.dot(h2, w2_b[...], preferred_element_type=jnp.float32)
            + b_b[...]
        )

    row_spec = pl.BlockSpec((R_BLK, D), lambda i: (i, 0))
    w_spec = pl.BlockSpec((D, D), lambda i: (0, 0))
    b_spec = pl.BlockSpec((1, D), lambda i: (0, 0))
    return pl.pallas_call(
        body,
        grid=(N // R_BLK,),
        in_specs=[row_spec, row_spec, row_spec, w_spec, b_spec],
        out_specs=row_spec,
        out_shape=jax.ShapeDtypeStruct((N, D), jnp.float32),
    )(acc, q0, q1, w2, bias)


def kernel(x, edge_index, edge_weight, shared_weight, output_bias):
    src = edge_index[1].astype(jnp.int32)
    dst = edge_index[0].astype(jnp.int32)
    w = edge_weight.astype(jnp.float32)
    pad = E_PAD - E
    src = jnp.concatenate([src, jnp.zeros((pad,), jnp.int32)])
    dst = jnp.concatenate([dst, jnp.zeros((pad,), jnp.int32)])
    w = jnp.concatenate([w, jnp.zeros((pad,), jnp.float32)])

    w0 = shared_weight[:D]
    w1 = shared_weight[D:2 * D]
    w2 = shared_weight[2 * D:]
    bias = output_bias.reshape(1, D)

    p = _spmm_sc(x, src, dst, w)
    h1, acc = _fuse1(x, p[0], p[1], w0, w1)
    q = _spmm_sc(h1, src, dst, w)
    return _fuse2(acc, q[0], q[1], w2, bias)


# speedup vs baseline: 1.2676x; 1.0779x over previous
import functools

import jax
import jax.numpy as jnp
from jax import lax
from jax.experimental import pallas as pl
from jax.experimental.pallas import tpu as pltpu
from jax.experimental.pallas import tpu_sc as plsc

N = 10000
D = 128
E = 320000
C = 128
NC = 2
NS = 16
NW = NC * NS
CHUNKS_TOTAL = -(-E // (C * NW))
E_PAD = CHUNKS_TOTAL * C * NW
PER_W = E_PAD // NW
CHUNKS = PER_W // C
N_PAD = 10240
RPT = N_PAD // NS


def _spmm_sc(h, sd3, w):
    mesh = plsc.VectorSubcoreMesh(core_axis_name="c", subcore_axis_name="s")

    @functools.partial(
        pl.kernel,
        out_type=jax.ShapeDtypeStruct((NC, N_PAD, D), jnp.float32),
        mesh=mesh,
        scratch_types=[
            pltpu.VMEM((2, C), jnp.int32),
            pltpu.VMEM((C + 16,), jnp.float32),
            pltpu.VMEM((C, D), jnp.float32),
            pltpu.VMEM_SHARED((N_PAD, D), jnp.float32),
            pltpu.SemaphoreType.DMA,
        ],
    )
    def k(h_hbm, sd_hbm, w_hbm, out_hbm,
          sd_v, w_v, rows_v, accum, sem):
        cid = lax.axis_index("c")
        sid = lax.axis_index("s")
        wid = sid * NC + cid

        def zrow(r, _):
            for j in range(D // 16):
                rows_v[r, pl.ds(j * 16, 16)] = jnp.zeros((16,), jnp.float32)
            return 0
        lax.fori_loop(0, C, zrow, 0)
        base = sid * RPT
        for kblk in range(RPT // C):
            pltpu.sync_copy(rows_v, accum.at[pl.ds(base + kblk * C, C)])
        plsc.subcore_barrier()

        ebase = wid * PER_W
        cb = wid * CHUNKS

        def chunk_body(ci, _):
            off = ebase + ci * C
            pltpu.sync_copy(sd_hbm.at[cb + ci], sd_v)
            pltpu.sync_copy(w_hbm.at[pl.ds(off, C)], w_v.at[pl.ds(0, C)])
            pltpu.async_copy(h_hbm.at[sd_v.at[0]], rows_v, sem).wait()

            def row_body(r, _):
                ws = w_v[pl.ds(r, 16)][0]
                for j in range(D // 16):
                    sl = pl.ds(j * 16, 16)
                    rows_v[r, sl] = rows_v[r, sl] * ws
                return 0
            lax.fori_loop(0, C, row_body, 0)

            pltpu.sync_copy(rows_v, accum.at[sd_v.at[1]], add=True)
            return 0

        lax.fori_loop(0, CHUNKS, chunk_body, 0)
        plsc.subcore_barrier()

        pltpu.sync_copy(accum.at[pl.ds(base, RPT)],
                        out_hbm.at[cid, pl.ds(base, RPT)])

    return k(h, sd3, w)


R_BLK = 1000


def _fuse1(x, p0, p1, w0, w1):
    def body(x_b, p0_b, p1_b, w0_b, w1_b, h1_b, acc_b):
        h1 = p0_b[...] + p1_b[...]
        h1_b[...] = h1
        acc_b[...] = (
            jnp.dot(x_b[...], w0_b[...], preferred_element_type=jnp.float32)
            + jnp.dot(h1, w1_b[...], preferred_element_type=jnp.float32)
        )

    row_spec = pl.BlockSpec((R_BLK, D), lambda i: (i, 0))
    w_spec = pl.BlockSpec((D, D), lambda i: (0, 0))
    return pl.pallas_call(
        body,
        grid=(N // R_BLK,),
        in_specs=[row_spec, row_spec, row_spec, w_spec, w_spec],
        out_specs=[row_spec, row_spec],
        out_shape=[
            jax.ShapeDtypeStruct((N, D), jnp.float32),
            jax.ShapeDtypeStruct((N, D), jnp.float32),
        ],
    )(x, p0, p1, w0, w1)


def _fuse2(acc, q0, q1, w2, bias):
    def body(acc_b, q0_b, q1_b, w2_b, b_b, out_b):
        h2 = q0_b[...] + q1_b[...]
        out_b[...] = (
            acc_b[...]
            + jnp.dot(h2, w2_b[...], preferred_element_type=jnp.float32)
            + b_b[...]
        )

    row_spec = pl.BlockSpec((R_BLK, D), lambda i: (i, 0))
    w_spec = pl.BlockSpec((D, D), lambda i: (0, 0))
    b_spec = pl.BlockSpec((1, D), lambda i: (0, 0))
    return pl.pallas_call(
        body,
        grid=(N // R_BLK,),
        in_specs=[row_spec, row_spec, row_spec, w_spec, b_spec],
        out_specs=row_spec,
        out_shape=jax.ShapeDtypeStruct((N, D), jnp.float32),
    )(acc, q0, q1, w2, bias)


def kernel(x, edge_index, edge_weight, shared_weight, output_bias):
    src = edge_index[1].astype(jnp.int32)
    dst = edge_index[0].astype(jnp.int32)
    w = edge_weight.astype(jnp.float32)
    pad = E_PAD - E
    src = jnp.concatenate([src, jnp.zeros((pad,), jnp.int32)])
    dst = jnp.concatenate([dst, jnp.zeros((pad,), jnp.int32)])
    w = jnp.concatenate([w, jnp.zeros((pad,), jnp.float32)])
    sd3 = jnp.stack([src.reshape(-1, C), dst.reshape(-1, C)], axis=1)

    w0 = shared_weight[:D]
    w1 = shared_weight[D:2 * D]
    w2 = shared_weight[2 * D:]
    bias = output_bias.reshape(1, D)

    p = _spmm_sc(x, sd3, w)
    h1, acc = _fuse1(x, p[0], p[1], w0, w1)
    q = _spmm_sc(h1, sd3, w)
    return _fuse2(acc, q[0], q[1], w2, bias)


# speedup vs baseline: 1.3729x; 1.0830x over previous
import functools

import jax
import jax.numpy as jnp
from jax import lax
from jax.experimental import pallas as pl
from jax.experimental.pallas import tpu as pltpu
from jax.experimental.pallas import tpu_sc as plsc

N = 10000
D = 128
E = 320000
C = 128
NC = 2
NS = 16
NW = NC * NS
CHUNKS_TOTAL = -(-E // (C * NW))
E_PAD = CHUNKS_TOTAL * C * NW
PER_W = E_PAD // NW
CHUNKS = PER_W // C
N_PAD = 10240
RPT = N_PAD // NS


def _spmm_sc(h, sd3):
    mesh = plsc.VectorSubcoreMesh(core_axis_name="c", subcore_axis_name="s")

    @functools.partial(
        pl.kernel,
        out_type=jax.ShapeDtypeStruct((NC, N_PAD, D), jnp.float32),
        mesh=mesh,
        scratch_types=[
            pltpu.VMEM((4, C), jnp.int32),
            pltpu.VMEM((C, D), jnp.float32),
            pltpu.VMEM_SHARED((N_PAD, D), jnp.float32),
            pltpu.SemaphoreType.DMA,
        ],
    )
    def k(h_hbm, sd_hbm, out_hbm,
          sd_v, rows_v, accum, sem):
        cid = lax.axis_index("c")
        sid = lax.axis_index("s")
        wid = sid * NC + cid

        def zrow(r, _):
            for j in range(D // 16):
                rows_v[r, pl.ds(j * 16, 16)] = jnp.zeros((16,), jnp.float32)
            return 0
        lax.fori_loop(0, C, zrow, 0)
        base = sid * RPT
        for kblk in range(RPT // C):
            pltpu.sync_copy(rows_v, accum.at[pl.ds(base + kblk * C, C)])
        plsc.subcore_barrier()

        cb = wid * CHUNKS

        def chunk_body(ci, _):
            pltpu.sync_copy(sd_hbm.at[cb + ci], sd_v.at[pl.ds(0, 3)])
            pltpu.async_copy(h_hbm.at[sd_v.at[0]], rows_v, sem).wait()

            def row_body(r, _):
                ws = jax.lax.bitcast_convert_type(
                    sd_v[2, pl.ds(r, 16)], jnp.float32)[0]
                for j in range(D // 16):
                    sl = pl.ds(j * 16, 16)
                    rows_v[r, sl] = rows_v[r, sl] * ws
                return 0
            lax.fori_loop(0, C, row_body, 0)

            pltpu.sync_copy(rows_v, accum.at[sd_v.at[1]], add=True)
            return 0

        lax.fori_loop(0, CHUNKS, chunk_body, 0)
        plsc.subcore_barrier()

        pltpu.sync_copy(accum.at[pl.ds(base, RPT)],
                        out_hbm.at[cid, pl.ds(base, RPT)])

    return k(h, sd3)


R_BLK = 1000


def _fuse1(x, p0, p1, w0, w1):
    def body(x_b, p0_b, p1_b, w0_b, w1_b, h1_b, acc_b):
        h1 = p0_b[...] + p1_b[...]
        h1_b[...] = h1
        acc_b[...] = (
            jnp.dot(x_b[...], w0_b[...], preferred_element_type=jnp.float32)
            + jnp.dot(h1, w1_b[...], preferred_element_type=jnp.float32)
        )

    row_spec = pl.BlockSpec((R_BLK, D), lambda i: (i, 0))
    w_spec = pl.BlockSpec((D, D), lambda i: (0, 0))
    return pl.pallas_call(
        body,
        grid=(N // R_BLK,),
        in_specs=[row_spec, row_spec, row_spec, w_spec, w_spec],
        out_specs=[row_spec, row_spec],
        out_shape=[
            jax.ShapeDtypeStruct((N, D), jnp.float32),
            jax.ShapeDtypeStruct((N, D), jnp.float32),
        ],
    )(x, p0, p1, w0, w1)


def _fuse2(acc, q0, q1, w2, bias):
    def body(acc_b, q0_b, q1_b, w2_b, b_b, out_b):
        h2 = q0_b[...] + q1_b[...]
        out_b[...] = (
            acc_b[...]
            + jnp.dot(h2, w2_b[...], preferred_element_type=jnp.float32)
            + b_b[...]
        )

    row_spec = pl.BlockSpec((R_BLK, D), lambda i: (i, 0))
    w_spec = pl.BlockSpec((D, D), lambda i: (0, 0))
    b_spec = pl.BlockSpec((1, D), lambda i: (0, 0))
    return pl.pallas_call(
        body,
        grid=(N // R_BLK,),
        in_specs=[row_spec, row_spec, row_spec, w_spec, b_spec],
        out_specs=row_spec,
        out_shape=jax.ShapeDtypeStruct((N, D), jnp.float32),
    )(acc, q0, q1, w2, bias)


def kernel(x, edge_index, edge_weight, shared_weight, output_bias):
    src = edge_index[1].astype(jnp.int32)
    dst = edge_index[0].astype(jnp.int32)
    w = edge_weight.astype(jnp.float32)
    pad = E_PAD - E
    src = jnp.concatenate([src, jnp.zeros((pad,), jnp.int32)])
    dst = jnp.concatenate([dst, jnp.zeros((pad,), jnp.int32)])
    w = jnp.concatenate([w, jnp.zeros((pad,), jnp.float32)])
    wbits = jax.lax.bitcast_convert_type(w.reshape(-1, C), jnp.int32)
    sd3 = jnp.stack([src.reshape(-1, C), dst.reshape(-1, C), wbits], axis=1)

    w0 = shared_weight[:D]
    w1 = shared_weight[D:2 * D]
    w2 = shared_weight[2 * D:]
    bias = output_bias.reshape(1, D)

    p = _spmm_sc(x, sd3)
    h1, acc = _fuse1(x, p[0], p[1], w0, w1)
    q = _spmm_sc(h1, sd3)
    return _fuse2(acc, q[0], q[1], w2, bias)
